# Initial kernel scaffold; baseline (speedup 1.0000x reference)
import jax
import jax.numpy as jnp
from jax.experimental import pallas as pl


def kernel(node_features, density_features, edge_features, edge_index, basis, W_rad1, b_rad1, W_rad2, b_rad2, W_conv, W_self, W_norm, b_norm, ln_g, ln_b, W_lin):
    raise NotImplementedError("write your pallas kernel here")


# speedup vs baseline: 1.9177x; 1.9177x over previous
import functools

import jax
import jax.numpy as jnp
from jax import lax
from jax.experimental import pallas as pl
from jax.experimental.pallas import tpu as pltpu
from jax.experimental.pallas import tpu_sc as plsc

N = 10000
E = 320000
C = 128
DE = 16
H = 32

NC = 2
NS = 16
NW = NC * NS

K = 128
T = 10112
E_PAD = T * NW
NCHUNK = T // K
N_PAD = 10240
ROWS_PER_TILE = N_PAD // NS

BE = 4096


def _radial_body(ef_ref, basis_ref, w1_ref, b1_ref, w2_ref, b2_ref, out_ref):
    h = jnp.maximum(
        jnp.dot(ef_ref[...], w1_ref[...], preferred_element_type=jnp.float32)
        + b1_ref[...][None, :], 0.0)
    r = (jnp.dot(h, w2_ref[...], preferred_element_type=jnp.float32)
         + b2_ref[...][None, :])
    out_ref[...] = r * basis_ref[...]


def _radial(ef_pad, basis_pad, w1, b1, w2, b2):
    grid = E_PAD // BE
    return pl.pallas_call(
        _radial_body,
        grid=(grid,),
        in_specs=[
            pl.BlockSpec((BE, DE), lambda i: (i, 0)),
            pl.BlockSpec((BE, 1), lambda i: (i, 0)),
            pl.BlockSpec((DE, H), lambda i: (0, 0)),
            pl.BlockSpec((H,), lambda i: (0,)),
            pl.BlockSpec((H, C), lambda i: (0, 0)),
            pl.BlockSpec((C,), lambda i: (0,)),
        ],
        out_specs=pl.BlockSpec((BE, C), lambda i: (i, 0)),
        out_shape=jax.ShapeDtypeStruct((E_PAD, C), jnp.float32),
    )(ef_pad, basis_pad, w1, b1, w2, b2)


def _scatter_body(x_hbm, rb_hbm, src_hbm, dst_hbm, out_hbm,
                  idx_src, idx_dst, xg, rbv, agg_sh, sem):
    c = lax.axis_index("c")
    s = lax.axis_index("s")
    wid = c * NS + s

    def zrow(i, _):
        for j in range(C // 16):
            xg[i, pl.ds(j * 16, 16)] = jnp.zeros((16,), jnp.float32)
        return _
    lax.fori_loop(0, K, zrow, None)
    row0 = s * ROWS_PER_TILE
    for z in range(ROWS_PER_TILE // K):
        pltpu.sync_copy(xg, agg_sh.at[pl.ds(row0 + z * K, K)])
    plsc.subcore_barrier()

    def chunk(ci, _):
        base = wid * T + ci * K
        pltpu.sync_copy(src_hbm.at[pl.ds(base, K)], idx_src)
        pltpu.sync_copy(dst_hbm.at[pl.ds(base, K)], idx_dst)
        pltpu.async_copy(x_hbm.at[idx_src], xg, sem).wait()
        pltpu.sync_copy(rb_hbm.at[pl.ds(base, K)], rbv)

        def mrow(i, _):
            for j in range(C // 16):
                sl = pl.ds(j * 16, 16)
                rbv[i, sl] = rbv[i, sl] * xg[i, sl]
            return _
        lax.fori_loop(0, K, mrow, None)
        pltpu.sync_copy(rbv, agg_sh.at[idx_dst], add=True)
        return _
    lax.fori_loop(0, NCHUNK, chunk, None)

    plsc.subcore_barrier()
    pltpu.sync_copy(agg_sh.at[pl.ds(row0, ROWS_PER_TILE)],
                    out_hbm.at[c, pl.ds(row0, ROWS_PER_TILE)])


def _sc_scatter(x, rb, src_pad, dst_pad):
    mesh = plsc.VectorSubcoreMesh(core_axis_name="c", subcore_axis_name="s")
    f = pl.kernel(
        _scatter_body,
        out_type=jax.ShapeDtypeStruct((NC, N_PAD, C), jnp.float32),
        mesh=mesh,
        scratch_types=[
            pltpu.VMEM((K,), jnp.int32),
            pltpu.VMEM((K,), jnp.int32),
            pltpu.VMEM((K, C), jnp.float32),
            pltpu.VMEM((K, C), jnp.float32),
            pltpu.VMEM_SHARED((N_PAD, C), jnp.float32),
            pltpu.SemaphoreType.DMA,
        ],
    )
    return f(x, rb, src_pad, dst_pad)


def _node_body(p_ref, x_ref, dens_ref, wc_ref, ws_ref, wn_ref, bn_ref,
               g_ref, b_ref, wl_ref, out_ref):
    agg = p_ref[0] + p_ref[1]
    u = (jnp.dot(agg, wc_ref[...], preferred_element_type=jnp.float32)
         + jnp.dot(x_ref[...], ws_ref[...], preferred_element_type=jnp.float32))
    norm = jnp.abs(u) + 1e-6
    phase = u / norm
    mu = jnp.mean(norm, axis=-1, keepdims=True)
    var = jnp.mean((norm - mu) ** 2, axis=-1, keepdims=True)
    nln = (norm - mu) * lax.rsqrt(var + 1e-5) * g_ref[...][None, :] \
        + b_ref[...][None, :]
    t = jnp.maximum(
        jnp.dot(nln, wn_ref[...], preferred_element_type=jnp.float32)
        + bn_ref[...][None, :], 0.0)
    upd = jnp.dot(t * phase, wl_ref[...], preferred_element_type=jnp.float32)
    out_ref[...] = dens_ref[...] + upd


def _node_pipeline(partial, x, dens, wc, ws, wn, bn, g, b, wl):
    BN = 1000
    grid = N // BN
    return pl.pallas_call(
        _node_body,
        grid=(grid,),
        in_specs=[
            pl.BlockSpec((NC, BN, C), lambda i: (0, i, 0)),
            pl.BlockSpec((BN, C), lambda i: (i, 0)),
            pl.BlockSpec((BN, C), lambda i: (i, 0)),
            pl.BlockSpec((C, C), lambda i: (0, 0)),
            pl.BlockSpec((C, C), lambda i: (0, 0)),
            pl.BlockSpec((C, C), lambda i: (0, 0)),
            pl.BlockSpec((C,), lambda i: (0,)),
            pl.BlockSpec((C,), lambda i: (0,)),
            pl.BlockSpec((C,), lambda i: (0,)),
            pl.BlockSpec((C, C), lambda i: (0, 0)),
        ],
        out_specs=pl.BlockSpec((BN, C), lambda i: (i, 0)),
        out_shape=jax.ShapeDtypeStruct((N, C), jnp.float32),
    )(partial, x, dens, wc, ws, wn, bn, g, b, wl)


def kernel(node_features, density_features, edge_features, edge_index, basis,
           W_rad1, b_rad1, W_rad2, b_rad2, W_conv, W_self,
           W_norm, b_norm, ln_g, ln_b, W_lin):
    pad = E_PAD - E
    ef_pad = jnp.pad(edge_features, ((0, pad), (0, 0)))
    basis_pad = jnp.pad(basis, ((0, pad), (0, 0)))
    src_pad = jnp.pad(edge_index[0], (0, pad))
    dst_pad = jnp.pad(edge_index[1], (0, pad), constant_values=N_PAD - 1)

    rb = _radial(ef_pad, basis_pad, W_rad1, b_rad1, W_rad2, b_rad2)
    partial = _sc_scatter(node_features, rb, src_pad, dst_pad)
    out = _node_pipeline(partial[:, :N, :], node_features, density_features,
                         W_conv, W_self, W_norm, b_norm, ln_g, ln_b, W_lin)
    return out


# speedup vs baseline: 3.8131x; 1.9884x over previous
import jax
import jax.numpy as jnp
from jax import lax
from jax.experimental import pallas as pl
from jax.experimental.pallas import tpu as pltpu
from jax.experimental.pallas import tpu_sc as plsc

N = 10000
E = 320000
C = 128
DE = 16
H = 32

NC = 2
NS = 16
NW = NC * NS

T = E // NW
K = 64
NFULL = T // K
KT = T - NFULL * K
N_PAD = 10240
ROWS_PER_TILE = N_PAD // NS

BE = 6400


def _radial_body(ef_ref, basis_ref, w1_ref, b1_ref, w2_ref, b2_ref, out_ref):
    h = jnp.maximum(
        jnp.dot(ef_ref[...], w1_ref[...], preferred_element_type=jnp.float32)
        + b1_ref[...][None, :], 0.0)
    r = (jnp.dot(h, w2_ref[...], preferred_element_type=jnp.float32)
         + b2_ref[...][None, :])
    out_ref[...] = r * basis_ref[...]


def _radial(ef, basis, w1, b1, w2, b2):
    return pl.pallas_call(
        _radial_body,
        grid=(E // BE,),
        in_specs=[
            pl.BlockSpec((BE, DE), lambda i: (i, 0)),
            pl.BlockSpec((BE, 1), lambda i: (i, 0)),
            pl.BlockSpec((DE, H), lambda i: (0, 0)),
            pl.BlockSpec((H,), lambda i: (0,)),
            pl.BlockSpec((H, C), lambda i: (0, 0)),
            pl.BlockSpec((C,), lambda i: (0,)),
        ],
        out_specs=pl.BlockSpec((BE, C), lambda i: (i, 0)),
        out_shape=jax.ShapeDtypeStruct((E, C), jnp.float32),
    )(ef, basis, w1, b1, w2, b2)


def _mul_rows(rbv, xg, nrows):
    def mrow(i, _):
        for j in range(C // 16):
            sl = pl.ds(j * 16, 16)
            rbv[i, sl] = rbv[i, sl] * xg[i, sl]
        return _
    lax.fori_loop(0, nrows, mrow, None)


def _scatter_body(x_hbm, rb_hbm, src_hbm, dst_hbm, out_hbm,
                  isrc_all, idst0, idst1, xg0, xg1, rbv0, rbv1,
                  idst_t, xg_t, rbv_t, agg_sh,
                  sem_g0, sem_g1, sem_r0, sem_r1, sem_i0, sem_i1):
    c = lax.axis_index("c")
    s = lax.axis_index("s")
    wid = c * NS + s
    tbase = wid * T

    idst = (idst0, idst1)
    xg = (xg0, xg1)
    rbv = (rbv0, rbv1)
    sem_g = (sem_g0, sem_g1)
    sem_r = (sem_r0, sem_r1)
    sem_i = (sem_i0, sem_i1)

    def zrow(i, _):
        for j in range(C // 16):
            xg0[i, pl.ds(j * 16, 16)] = jnp.zeros((16,), jnp.float32)
        return _
    lax.fori_loop(0, K, zrow, None)
    row0 = s * ROWS_PER_TILE
    for z in range(ROWS_PER_TILE // K):
        pltpu.sync_copy(xg0, agg_sh.at[pl.ds(row0 + z * K, K)])
    plsc.subcore_barrier()

    pltpu.sync_copy(src_hbm.at[pl.ds(tbase, T)], isrc_all)

    def start(ci, b):
        base = tbase + ci * K
        pltpu.async_copy(dst_hbm.at[pl.ds(base, K)], idst[b], sem_i[b])
        pltpu.async_copy(rb_hbm.at[pl.ds(base, K)], rbv[b], sem_r[b])
        pltpu.async_copy(x_hbm.at[isrc_all.at[pl.ds(ci * K, K)]], xg[b],
                         sem_g[b])

    def finish(b):
        pltpu.make_async_copy(rb_hbm.at[pl.ds(0, K)], rbv[b], sem_r[b]).wait()
        pltpu.make_async_copy(rb_hbm.at[pl.ds(0, K)], xg[b], sem_g[b]).wait()
        _mul_rows(rbv[b], xg[b], K)
        pltpu.make_async_copy(dst_hbm.at[pl.ds(0, K)], idst[b], sem_i[b]).wait()
        pltpu.sync_copy(rbv[b], agg_sh.at[idst[b]], add=True)

    start(0, 0)
    start(1, 1)

    def pair(i, _):
        c0 = 2 * i
        finish(0)

        @pl.when(c0 + 2 < NFULL)
        def _s0():
            start(c0 + 2, 0)
        finish(1)

        @pl.when(c0 + 3 < NFULL)
        def _s1():
            start(c0 + 3, 1)
        return _
    lax.fori_loop(0, NFULL // 2, pair, None)

    tb = tbase + NFULL * K
    pltpu.sync_copy(dst_hbm.at[pl.ds(tb, KT)], idst_t)
    pltpu.async_copy(x_hbm.at[isrc_all.at[pl.ds(NFULL * K, KT)]], xg_t,
                     sem_g0).wait()
    pltpu.sync_copy(rb_hbm.at[pl.ds(tb, KT)], rbv_t)
    _mul_rows(rbv_t, xg_t, KT)
    pltpu.sync_copy(rbv_t, agg_sh.at[idst_t], add=True)

    plsc.subcore_barrier()
    pltpu.sync_copy(agg_sh.at[pl.ds(row0, ROWS_PER_TILE)],
                    out_hbm.at[c, pl.ds(row0, ROWS_PER_TILE)])


def _sc_scatter(x, rb, src, dst):
    mesh = plsc.VectorSubcoreMesh(core_axis_name="c", subcore_axis_name="s")
    f = pl.kernel(
        _scatter_body,
        out_type=jax.ShapeDtypeStruct((NC, N_PAD, C), jnp.float32),
        mesh=mesh,
        scratch_types=[
            pltpu.VMEM((T,), jnp.int32),
            pltpu.VMEM((K,), jnp.int32),
            pltpu.VMEM((K,), jnp.int32),
            pltpu.VMEM((K, C), jnp.float32),
            pltpu.VMEM((K, C), jnp.float32),
            pltpu.VMEM((K, C), jnp.float32),
            pltpu.VMEM((K, C), jnp.float32),
            pltpu.VMEM((KT,), jnp.int32),
            pltpu.VMEM((KT, C), jnp.float32),
            pltpu.VMEM((KT, C), jnp.float32),
            pltpu.VMEM_SHARED((N_PAD, C), jnp.float32),
            pltpu.SemaphoreType.DMA,
            pltpu.SemaphoreType.DMA,
            pltpu.SemaphoreType.DMA,
            pltpu.SemaphoreType.DMA,
            pltpu.SemaphoreType.DMA,
            pltpu.SemaphoreType.DMA,
        ],
    )
    return f(x, rb, src, dst)


def _node_body(p_ref, x_ref, dens_ref, wc_ref, ws_ref, wn_ref, bn_ref,
               g_ref, b_ref, wl_ref, out_ref):
    agg = p_ref[0] + p_ref[1]
    u = (jnp.dot(agg, wc_ref[...], preferred_element_type=jnp.float32)
         + jnp.dot(x_ref[...], ws_ref[...], preferred_element_type=jnp.float32))
    norm = jnp.abs(u) + 1e-6
    phase = u / norm
    mu = jnp.mean(norm, axis=-1, keepdims=True)
    var = jnp.mean((norm - mu) ** 2, axis=-1, keepdims=True)
    nln = (norm - mu) * lax.rsqrt(var + 1e-5) * g_ref[...][None, :] \
        + b_ref[...][None, :]
    t = jnp.maximum(
        jnp.dot(nln, wn_ref[...], preferred_element_type=jnp.float32)
        + bn_ref[...][None, :], 0.0)
    upd = jnp.dot(t * phase, wl_ref[...], preferred_element_type=jnp.float32)
    out_ref[...] = dens_ref[...] + upd


def _node_pipeline(partial, x, dens, wc, ws, wn, bn, g, b, wl):
    BN = 1000
    return pl.pallas_call(
        _node_body,
        grid=(N // BN,),
        in_specs=[
            pl.BlockSpec((NC, BN, C), lambda i: (0, i, 0)),
            pl.BlockSpec((BN, C), lambda i: (i, 0)),
            pl.BlockSpec((BN, C), lambda i: (i, 0)),
            pl.BlockSpec((C, C), lambda i: (0, 0)),
            pl.BlockSpec((C, C), lambda i: (0, 0)),
            pl.BlockSpec((C, C), lambda i: (0, 0)),
            pl.BlockSpec((C,), lambda i: (0,)),
            pl.BlockSpec((C,), lambda i: (0,)),
            pl.BlockSpec((C,), lambda i: (0,)),
            pl.BlockSpec((C, C), lambda i: (0, 0)),
        ],
        out_specs=pl.BlockSpec((BN, C), lambda i: (i, 0)),
        out_shape=jax.ShapeDtypeStruct((N, C), jnp.float32),
    )(partial, x, dens, wc, ws, wn, bn, g, b, wl)


def kernel(node_features, density_features, edge_features, edge_index, basis,
           W_rad1, b_rad1, W_rad2, b_rad2, W_conv, W_self,
           W_norm, b_norm, ln_g, ln_b, W_lin):
    src = edge_index[0]
    dst = edge_index[1]
    rb = _radial(edge_features, basis, W_rad1, b_rad1, W_rad2, b_rad2)
    partial = _sc_scatter(node_features, rb, src, dst)
    return _node_pipeline(partial, node_features, density_features,
                          W_conv, W_self, W_norm, b_norm, ln_g, ln_b, W_lin)
